# baseline (speedup 1.0000x reference)
import functools

import jax
import jax.numpy as jnp
from jax import lax
from jax.experimental import pallas as pl
from jax.experimental.pallas import tpu as pltpu
from jax.experimental.pallas import tpu_sc as plsc

N = 10000
E = 320000
NPAD = 10240
NW = 32
EPW = E // NW
K = 64
EPW_PAD = 10240
NCHUNK = EPW_PAD // K
GP = K // 16
RPT = NPAD // 16
_SC_PARAMS = pltpu.CompilerParams(use_tc_tiling_on_sc=False)
_MESH = dict(core_axis_name="c", subcore_axis_name="s")


@functools.partial(
    pl.kernel,
    out_type=jax.ShapeDtypeStruct((2, NPAD, 16), jnp.float32),
    mesh=plsc.VectorSubcoreMesh(**_MESH),
    compiler_params=_SC_PARAMS,
    scratch_types=[
        pltpu.VMEM((NCHUNK, K), jnp.int32),
        pltpu.VMEM((K, 16), jnp.float32),
        pltpu.VMEM_SHARED((NPAD, 16), jnp.float32),
    ],
)
def _deg(dstm_hbm, ones_hbm, zeros_hbm, out_hbm, dst_v, ones_v, acc_sh):
    c = lax.axis_index("c")
    s = lax.axis_index("s")
    wid = s * 2 + c
    pltpu.sync_copy(dstm_hbm.at[wid], dst_v)
    pltpu.sync_copy(ones_hbm, ones_v)
    pltpu.sync_copy(zeros_hbm.at[pl.ds(s * RPT, RPT)],
                    acc_sh.at[pl.ds(s * RPT, RPT)])
    plsc.subcore_barrier()

    def body(j, carry):
        pltpu.sync_copy(ones_v, acc_sh.at[dst_v.at[j]], add=True)
        return carry

    lax.fori_loop(0, NCHUNK, body, 0)
    plsc.subcore_barrier()
    pltpu.sync_copy(acc_sh.at[pl.ds(s * RPT, RPT)],
                    out_hbm.at[c, pl.ds(s * RPT, RPT)])


def _make_prop(F):
    @functools.partial(
        pl.kernel,
        out_type=jax.ShapeDtypeStruct((2, NPAD, F), jnp.float32),
        mesh=plsc.VectorSubcoreMesh(**_MESH),
        compiler_params=_SC_PARAMS,
        scratch_types=[
            pltpu.VMEM((NCHUNK, K), jnp.int32),
            pltpu.VMEM((NCHUNK, K), jnp.int32),
            pltpu.VMEM((K, F), jnp.float32),
            pltpu.VMEM((K, F), jnp.float32),
            pltpu.VMEM_SHARED((NPAD, F), jnp.float32),
            pltpu.SemaphoreType.DMA,
            pltpu.SemaphoreType.DMA,
        ],
    )
    def prop(h_hbm, zeros_hbm, srcm_hbm, dstm_hbm, out_hbm,
             src_v, dst_v, rows0, rows1, acc_sh, sem0, sem1):
        c = lax.axis_index("c")
        s = lax.axis_index("s")
        wid = s * 2 + c
        pltpu.sync_copy(srcm_hbm.at[wid], src_v)
        pltpu.sync_copy(dstm_hbm.at[wid], dst_v)
        pltpu.sync_copy(zeros_hbm.at[pl.ds(s * RPT, RPT)],
                        acc_sh.at[pl.ds(s * RPT, RPT)])
        plsc.subcore_barrier()

        pltpu.async_copy(h_hbm.at[src_v.at[0]], rows0, sem0)

        def body(jj, carry):
            j0 = jj * 2
            j1 = j0 + 1
            j2 = j0 + 2
            pltpu.async_copy(h_hbm.at[src_v.at[j1]], rows1, sem1)
            pltpu.make_async_copy(h_hbm.at[src_v.at[j0]], rows0, sem0).wait()
            pltpu.sync_copy(rows0, acc_sh.at[dst_v.at[j0]], add=True)

            @pl.when(j2 < NCHUNK)
            def _():
                pltpu.async_copy(h_hbm.at[src_v.at[j2]], rows0, sem0)

            pltpu.make_async_copy(h_hbm.at[src_v.at[j1]], rows1, sem1).wait()
            pltpu.sync_copy(rows1, acc_sh.at[dst_v.at[j1]], add=True)
            return carry

        lax.fori_loop(0, NCHUNK // 2, body, 0)
        plsc.subcore_barrier()
        pltpu.sync_copy(acc_sh.at[pl.ds(s * RPT, RPT)],
                        out_hbm.at[c, pl.ds(s * RPT, RPT)])

    return prop


_prop64 = _make_prop(64)
_prop16 = _make_prop(16)

_B = 1000


def _mm1_body(x_ref, w_ref, d0_ref, d1_ref, h_ref, dinv_ref):
    deg = d0_ref[...] + d1_ref[...] + 1.0
    dinv = lax.rsqrt(deg)
    h = jnp.dot(x_ref[...], w_ref[...], preferred_element_type=jnp.float32)
    h_ref[...] = h * dinv
    dinv_ref[...] = dinv


def _mm1(x, W1, d0, d1):
    return pl.pallas_call(
        _mm1_body,
        grid=(N // _B,),
        in_specs=[
            pl.BlockSpec((_B, 128), lambda i: (i, 0)),
            pl.BlockSpec((128, 64), lambda i: (0, 0)),
            pl.BlockSpec((_B, 1), lambda i: (i, 0)),
            pl.BlockSpec((_B, 1), lambda i: (i, 0)),
        ],
        out_specs=[
            pl.BlockSpec((_B, 64), lambda i: (i, 0)),
            pl.BlockSpec((_B, 1), lambda i: (i, 0)),
        ],
        out_shape=[
            jax.ShapeDtypeStruct((N, 64), jnp.float32),
            jax.ShapeDtypeStruct((N, 1), jnp.float32),
        ],
    )(x, W1, d0, d1)


def _mm2_body(p0_ref, p1_ref, h_ref, dinv_ref, b1_ref, w2_ref, out_ref):
    dinv = dinv_ref[...]
    agg = p0_ref[...] + p1_ref[...] + h_ref[...]
    t = jnp.maximum(dinv * agg + b1_ref[...], 0.0)
    h2 = jnp.dot(t, w2_ref[...], preferred_element_type=jnp.float32)
    out_ref[...] = h2 * dinv


def _mm2(p0, p1, h1p, dinv, b1, W2):
    return pl.pallas_call(
        _mm2_body,
        grid=(N // _B,),
        in_specs=[
            pl.BlockSpec((_B, 64), lambda i: (i, 0)),
            pl.BlockSpec((_B, 64), lambda i: (i, 0)),
            pl.BlockSpec((_B, 64), lambda i: (i, 0)),
            pl.BlockSpec((_B, 1), lambda i: (i, 0)),
            pl.BlockSpec((1, 64), lambda i: (0, 0)),
            pl.BlockSpec((64, 16), lambda i: (0, 0)),
        ],
        out_specs=pl.BlockSpec((_B, 16), lambda i: (i, 0)),
        out_shape=jax.ShapeDtypeStruct((N, 16), jnp.float32),
    )(p0, p1, h1p, dinv, b1, W2)


def _final_body(q0_ref, q1_ref, h_ref, dinv_ref, b2_ref, out_ref):
    o = dinv_ref[...] * (q0_ref[...] + q1_ref[...] + h_ref[...]) + b2_ref[...]
    m = jnp.max(o, axis=1, keepdims=True)
    lse = jnp.log(jnp.sum(jnp.exp(o - m), axis=1, keepdims=True)) + m
    out_ref[...] = o - lse


def _final(q0, q1, h2p, dinv, b2):
    return pl.pallas_call(
        _final_body,
        grid=(N // _B,),
        in_specs=[
            pl.BlockSpec((_B, 16), lambda i: (i, 0)),
            pl.BlockSpec((_B, 16), lambda i: (i, 0)),
            pl.BlockSpec((_B, 16), lambda i: (i, 0)),
            pl.BlockSpec((_B, 1), lambda i: (i, 0)),
            pl.BlockSpec((1, 16), lambda i: (0, 0)),
        ],
        out_specs=pl.BlockSpec((_B, 16), lambda i: (i, 0)),
        out_shape=jax.ShapeDtypeStruct((N, 16), jnp.float32),
    )(q0, q1, h2p, dinv, b2)


def kernel(x, edge_index, W1, b1, W2, b2):
    pad = jnp.full((NW, EPW_PAD - EPW), N, jnp.int32)
    pad_d = jnp.full((NW, EPW_PAD - EPW), NPAD - 1, jnp.int32)
    src = jnp.concatenate(
        [edge_index[0].reshape(NW, EPW), pad], axis=1).reshape(NW, NCHUNK, K)
    dst_flat = jnp.concatenate(
        [edge_index[1].reshape(NW, EPW), pad_d], axis=1)
    dst = dst_flat.reshape(NW, NCHUNK, K)

    zeros64 = jnp.zeros((NPAD, 64), jnp.float32)
    zeros16 = jnp.zeros((NPAD, 16), jnp.float32)
    ones_k = jnp.ones((K, 16), jnp.float32)

    degp = _deg(dst, ones_k, zeros16)
    h1p, dinv = _mm1(x, W1, degp[0, :N, :1], degp[1, :N, :1])

    h1pad = jnp.concatenate(
        [h1p, jnp.zeros((NPAD - N, 64), jnp.float32)], axis=0)
    p = _prop64(h1pad, zeros64, src, dst)
    h2p = _mm2(p[0, :N], p[1, :N], h1p, dinv, b1.reshape(1, 64), W2)

    h2pad = jnp.concatenate(
        [h2p, jnp.zeros((NPAD - N, 16), jnp.float32)], axis=0)
    q = _prop16(h2pad, zeros16, src, dst)
    return _final(q[0, :N], q[1, :N], h2p, dinv, b2.reshape(1, 16))


# speedup vs baseline: 1.8778x; 1.8778x over previous
import functools

import jax
import jax.numpy as jnp
from jax import lax
from jax.experimental import pallas as pl
from jax.experimental.pallas import tpu as pltpu
from jax.experimental.pallas import tpu_sc as plsc

N = 10000
E = 320000
NPAD = 10240
NW = 32
EPW = E // NW
K = 128
EPW_PAD = 10240
NCHUNK = EPW_PAD // K
GP = K // 16
RPT = NPAD // 16
_SC_PARAMS = pltpu.CompilerParams(use_tc_tiling_on_sc=False)
_MESH = dict(core_axis_name="c", subcore_axis_name="s")


@functools.partial(
    pl.kernel,
    out_type=jax.ShapeDtypeStruct((2, NPAD, 16), jnp.float32),
    mesh=plsc.VectorSubcoreMesh(**_MESH),
    compiler_params=_SC_PARAMS,
    scratch_types=[
        pltpu.VMEM((NCHUNK, K), jnp.int32),
        pltpu.VMEM((K, 16), jnp.float32),
        pltpu.VMEM_SHARED((NPAD, 16), jnp.float32),
    ],
)
def _deg(dstm_hbm, ones_hbm, zeros_hbm, out_hbm, dst_v, ones_v, acc_sh):
    c = lax.axis_index("c")
    s = lax.axis_index("s")
    wid = s * 2 + c
    pltpu.sync_copy(dstm_hbm.at[wid], dst_v)
    pltpu.sync_copy(ones_hbm, ones_v)
    pltpu.sync_copy(zeros_hbm.at[pl.ds(s * RPT, RPT)],
                    acc_sh.at[pl.ds(s * RPT, RPT)])
    plsc.subcore_barrier()

    def body(j, carry):
        pltpu.sync_copy(ones_v, acc_sh.at[dst_v.at[j]], add=True)
        return carry

    lax.fori_loop(0, NCHUNK, body, 0)
    plsc.subcore_barrier()
    pltpu.sync_copy(acc_sh.at[pl.ds(s * RPT, RPT)],
                    out_hbm.at[c, pl.ds(s * RPT, RPT)])


def _make_prop(F):
    @functools.partial(
        pl.kernel,
        out_type=jax.ShapeDtypeStruct((2, NPAD, F), jnp.float32),
        mesh=plsc.VectorSubcoreMesh(**_MESH),
        compiler_params=_SC_PARAMS,
        scratch_types=[
            pltpu.VMEM((NCHUNK, K), jnp.int32),
            pltpu.VMEM((NCHUNK, K), jnp.int32),
            pltpu.VMEM((K, F), jnp.float32),
            pltpu.VMEM((K, F), jnp.float32),
            pltpu.VMEM_SHARED((NPAD, F), jnp.float32),
            pltpu.VMEM_SHARED((NPAD, F), jnp.float32),
            pltpu.SemaphoreType.DMA,
            pltpu.SemaphoreType.DMA,
        ],
    )
    def prop(h_hbm, zeros_hbm, srcm_hbm, dstm_hbm, out_hbm,
             src_v, dst_v, rows0, rows1, acc_sh, h_sh, sem0, sem1):
        c = lax.axis_index("c")
        s = lax.axis_index("s")
        wid = s * 2 + c
        pltpu.sync_copy(srcm_hbm.at[wid], src_v)
        pltpu.sync_copy(dstm_hbm.at[wid], dst_v)
        pltpu.sync_copy(zeros_hbm.at[pl.ds(s * RPT, RPT)],
                        acc_sh.at[pl.ds(s * RPT, RPT)])
        pltpu.sync_copy(h_hbm.at[pl.ds(s * RPT, RPT)],
                        h_sh.at[pl.ds(s * RPT, RPT)])
        plsc.subcore_barrier()

        pltpu.async_copy(h_sh.at[src_v.at[0]], rows0, sem0)

        def body(jj, carry):
            j0 = jj * 2
            j1 = j0 + 1
            j2 = j0 + 2
            pltpu.async_copy(h_sh.at[src_v.at[j1]], rows1, sem1)
            pltpu.make_async_copy(h_sh.at[src_v.at[j0]], rows0, sem0).wait()
            pltpu.sync_copy(rows0, acc_sh.at[dst_v.at[j0]], add=True)

            @pl.when(j2 < NCHUNK)
            def _():
                pltpu.async_copy(h_sh.at[src_v.at[j2]], rows0, sem0)

            pltpu.make_async_copy(h_sh.at[src_v.at[j1]], rows1, sem1).wait()
            pltpu.sync_copy(rows1, acc_sh.at[dst_v.at[j1]], add=True)
            return carry

        lax.fori_loop(0, NCHUNK // 2, body, 0)
        plsc.subcore_barrier()
        pltpu.sync_copy(acc_sh.at[pl.ds(s * RPT, RPT)],
                        out_hbm.at[c, pl.ds(s * RPT, RPT)])

    return prop


_prop64 = _make_prop(64)
_prop16 = _make_prop(16)

_B = 1000


def _mm1_body(x_ref, w_ref, d0_ref, d1_ref, h_ref, dinv_ref):
    deg = d0_ref[...] + d1_ref[...] + 1.0
    dinv = lax.rsqrt(deg)
    h = jnp.dot(x_ref[...], w_ref[...], preferred_element_type=jnp.float32)
    h_ref[...] = h * dinv
    dinv_ref[...] = dinv


def _mm1(x, W1, d0, d1):
    return pl.pallas_call(
        _mm1_body,
        grid=(N // _B,),
        in_specs=[
            pl.BlockSpec((_B, 128), lambda i: (i, 0)),
            pl.BlockSpec((128, 64), lambda i: (0, 0)),
            pl.BlockSpec((_B, 1), lambda i: (i, 0)),
            pl.BlockSpec((_B, 1), lambda i: (i, 0)),
        ],
        out_specs=[
            pl.BlockSpec((_B, 64), lambda i: (i, 0)),
            pl.BlockSpec((_B, 1), lambda i: (i, 0)),
        ],
        out_shape=[
            jax.ShapeDtypeStruct((N, 64), jnp.float32),
            jax.ShapeDtypeStruct((N, 1), jnp.float32),
        ],
    )(x, W1, d0, d1)


def _mm2_body(p0_ref, p1_ref, h_ref, dinv_ref, b1_ref, w2_ref, out_ref):
    dinv = dinv_ref[...]
    agg = p0_ref[...] + p1_ref[...] + h_ref[...]
    t = jnp.maximum(dinv * agg + b1_ref[...], 0.0)
    h2 = jnp.dot(t, w2_ref[...], preferred_element_type=jnp.float32)
    out_ref[...] = h2 * dinv


def _mm2(p0, p1, h1p, dinv, b1, W2):
    return pl.pallas_call(
        _mm2_body,
        grid=(N // _B,),
        in_specs=[
            pl.BlockSpec((_B, 64), lambda i: (i, 0)),
            pl.BlockSpec((_B, 64), lambda i: (i, 0)),
            pl.BlockSpec((_B, 64), lambda i: (i, 0)),
            pl.BlockSpec((_B, 1), lambda i: (i, 0)),
            pl.BlockSpec((1, 64), lambda i: (0, 0)),
            pl.BlockSpec((64, 16), lambda i: (0, 0)),
        ],
        out_specs=pl.BlockSpec((_B, 16), lambda i: (i, 0)),
        out_shape=jax.ShapeDtypeStruct((N, 16), jnp.float32),
    )(p0, p1, h1p, dinv, b1, W2)


def _final_body(q0_ref, q1_ref, h_ref, dinv_ref, b2_ref, out_ref):
    o = dinv_ref[...] * (q0_ref[...] + q1_ref[...] + h_ref[...]) + b2_ref[...]
    m = jnp.max(o, axis=1, keepdims=True)
    lse = jnp.log(jnp.sum(jnp.exp(o - m), axis=1, keepdims=True)) + m
    out_ref[...] = o - lse


def _final(q0, q1, h2p, dinv, b2):
    return pl.pallas_call(
        _final_body,
        grid=(N // _B,),
        in_specs=[
            pl.BlockSpec((_B, 16), lambda i: (i, 0)),
            pl.BlockSpec((_B, 16), lambda i: (i, 0)),
            pl.BlockSpec((_B, 16), lambda i: (i, 0)),
            pl.BlockSpec((_B, 1), lambda i: (i, 0)),
            pl.BlockSpec((1, 16), lambda i: (0, 0)),
        ],
        out_specs=pl.BlockSpec((_B, 16), lambda i: (i, 0)),
        out_shape=jax.ShapeDtypeStruct((N, 16), jnp.float32),
    )(q0, q1, h2p, dinv, b2)


def kernel(x, edge_index, W1, b1, W2, b2):
    pad = jnp.full((NW, EPW_PAD - EPW), N, jnp.int32)
    pad_d = jnp.full((NW, EPW_PAD - EPW), NPAD - 1, jnp.int32)
    src = jnp.concatenate(
        [edge_index[0].reshape(NW, EPW), pad], axis=1).reshape(NW, NCHUNK, K)
    dst_flat = jnp.concatenate(
        [edge_index[1].reshape(NW, EPW), pad_d], axis=1)
    dst = dst_flat.reshape(NW, NCHUNK, K)

    zeros64 = jnp.zeros((NPAD, 64), jnp.float32)
    zeros16 = jnp.zeros((NPAD, 16), jnp.float32)
    ones_k = jnp.ones((K, 16), jnp.float32)

    degp = _deg(dst, ones_k, zeros16)
    h1p, dinv = _mm1(x, W1, degp[0, :N, :1], degp[1, :N, :1])

    h1pad = jnp.concatenate(
        [h1p, jnp.zeros((NPAD - N, 64), jnp.float32)], axis=0)
    p = _prop64(h1pad, zeros64, src, dst)
    h2p = _mm2(p[0, :N], p[1, :N], h1p, dinv, b1.reshape(1, 64), W2)

    h2pad = jnp.concatenate(
        [h2p, jnp.zeros((NPAD - N, 16), jnp.float32)], axis=0)
    q = _prop16(h2pad, zeros16, src, dst)
    return _final(q[0, :N], q[1, :N], h2p, dinv, b2.reshape(1, 16))


# speedup vs baseline: 2.0377x; 1.0852x over previous
import functools

import jax
import jax.numpy as jnp
from jax import lax
from jax.experimental import pallas as pl
from jax.experimental.pallas import tpu as pltpu
from jax.experimental.pallas import tpu_sc as plsc

N = 10000
E = 320000
NPAD = 10240
NW = 32
EPW = E // NW
K = 128
EPW_PAD = 10240
NCHUNK = EPW_PAD // K
GP = K // 16
RPT = NPAD // 16
_SC_PARAMS = pltpu.CompilerParams(use_tc_tiling_on_sc=False)
_MESH = dict(core_axis_name="c", subcore_axis_name="s")


@functools.partial(
    pl.kernel,
    out_type=jax.ShapeDtypeStruct((2, NPAD, 16), jnp.float32),
    mesh=plsc.VectorSubcoreMesh(**_MESH),
    compiler_params=_SC_PARAMS,
    scratch_types=[
        pltpu.VMEM((NCHUNK, K), jnp.int32),
        pltpu.VMEM((K, 16), jnp.float32),
        pltpu.VMEM_SHARED((NPAD, 16), jnp.float32),
    ],
)
def _deg(dstm_hbm, ones_hbm, zeros_hbm, out_hbm, dst_v, ones_v, acc_sh):
    c = lax.axis_index("c")
    s = lax.axis_index("s")
    wid = s * 2 + c
    pltpu.sync_copy(dstm_hbm.at[wid], dst_v)
    pltpu.sync_copy(ones_hbm, ones_v)
    pltpu.sync_copy(zeros_hbm.at[pl.ds(s * RPT, RPT)],
                    acc_sh.at[pl.ds(s * RPT, RPT)])
    plsc.subcore_barrier()

    def body(j, carry):
        pltpu.sync_copy(ones_v, acc_sh.at[dst_v.at[j]], add=True)
        return carry

    lax.fori_loop(0, NCHUNK, body, 0)
    plsc.subcore_barrier()
    pltpu.sync_copy(acc_sh.at[pl.ds(s * RPT, RPT)],
                    out_hbm.at[c, pl.ds(s * RPT, RPT)])


def _make_prop(F):
    @functools.partial(
        pl.kernel,
        out_type=jax.ShapeDtypeStruct((2, NPAD, F), jnp.float32),
        mesh=plsc.VectorSubcoreMesh(**_MESH),
        compiler_params=_SC_PARAMS,
        scratch_types=[
            pltpu.VMEM((NCHUNK, K), jnp.int32),
            pltpu.VMEM((NCHUNK, K), jnp.int32),
            pltpu.VMEM((K, F), jnp.float32),
            pltpu.VMEM((K, F), jnp.float32),
            pltpu.VMEM_SHARED((NPAD, F), jnp.float32),
            pltpu.VMEM_SHARED((NPAD, F), jnp.float32),
            pltpu.SemaphoreType.DMA,
            pltpu.SemaphoreType.DMA,
        ],
    )
    def prop(h_hbm, zeros_hbm, srcm_hbm, dstm_hbm, out_hbm,
             src_v, dst_v, rows0, rows1, acc_sh, h_sh, sem0, sem1):
        c = lax.axis_index("c")
        s = lax.axis_index("s")
        wid = s * 2 + c
        pltpu.sync_copy(srcm_hbm.at[wid], src_v)
        pltpu.sync_copy(dstm_hbm.at[wid], dst_v)
        pltpu.sync_copy(zeros_hbm.at[pl.ds(s * RPT, RPT)],
                        acc_sh.at[pl.ds(s * RPT, RPT)])
        pltpu.sync_copy(h_hbm.at[pl.ds(s * RPT, RPT)],
                        h_sh.at[pl.ds(s * RPT, RPT)])
        plsc.subcore_barrier()

        pltpu.async_copy(h_sh.at[src_v.at[0]], rows0, sem0)

        def body(jj, carry):
            j0 = jj * 2
            j1 = j0 + 1
            j2 = j0 + 2
            pltpu.async_copy(h_sh.at[src_v.at[j1]], rows1, sem1)
            pltpu.make_async_copy(h_sh.at[src_v.at[j0]], rows0, sem0).wait()
            pltpu.sync_copy(rows0, acc_sh.at[dst_v.at[j0]], add=True)

            @pl.when(j2 < NCHUNK)
            def _():
                pltpu.async_copy(h_sh.at[src_v.at[j2]], rows0, sem0)

            pltpu.make_async_copy(h_sh.at[src_v.at[j1]], rows1, sem1).wait()
            pltpu.sync_copy(rows1, acc_sh.at[dst_v.at[j1]], add=True)
            return carry

        lax.fori_loop(0, NCHUNK // 2, body, 0)
        plsc.subcore_barrier()
        pltpu.sync_copy(acc_sh.at[pl.ds(s * RPT, RPT)],
                        out_hbm.at[c, pl.ds(s * RPT, RPT)])

    return prop


_prop64 = _make_prop(64)
_prop16 = _make_prop(16)

_B = 1000


def _mm1_body(x_ref, w_ref, d0_ref, d1_ref, h_ref, dinv_ref):
    deg = d0_ref[0, :, :1] + d1_ref[0, :, :1] + 1.0
    dinv = lax.rsqrt(deg)
    h = jnp.dot(x_ref[...], w_ref[...], preferred_element_type=jnp.float32)
    h_ref[...] = h * dinv
    dinv_ref[...] = dinv


def _mm1(x, W1, degp):
    return pl.pallas_call(
        _mm1_body,
        grid=(N // _B,),
        in_specs=[
            pl.BlockSpec((_B, 128), lambda i: (i, 0)),
            pl.BlockSpec((128, 64), lambda i: (0, 0)),
            pl.BlockSpec((1, _B, 16), lambda i: (0, i, 0)),
            pl.BlockSpec((1, _B, 16), lambda i: (1, i, 0)),
        ],
        out_specs=[
            pl.BlockSpec((_B, 64), lambda i: (i, 0)),
            pl.BlockSpec((_B, 1), lambda i: (i, 0)),
        ],
        out_shape=[
            jax.ShapeDtypeStruct((NPAD, 64), jnp.float32),
            jax.ShapeDtypeStruct((NPAD, 1), jnp.float32),
        ],
    )(x, W1, degp, degp)


def _mm2_body(p0_ref, p1_ref, h_ref, dinv_ref, b1_ref, w2_ref, out_ref):
    dinv = dinv_ref[...]
    agg = p0_ref[0] + p1_ref[0] + h_ref[...]
    t = jnp.maximum(dinv * agg + b1_ref[...], 0.0)
    h2 = jnp.dot(t, w2_ref[...], preferred_element_type=jnp.float32)
    out_ref[...] = h2 * dinv


def _mm2(p, h1p, dinv, b1, W2):
    return pl.pallas_call(
        _mm2_body,
        grid=(N // _B,),
        in_specs=[
            pl.BlockSpec((1, _B, 64), lambda i: (0, i, 0)),
            pl.BlockSpec((1, _B, 64), lambda i: (1, i, 0)),
            pl.BlockSpec((_B, 64), lambda i: (i, 0)),
            pl.BlockSpec((_B, 1), lambda i: (i, 0)),
            pl.BlockSpec((1, 64), lambda i: (0, 0)),
            pl.BlockSpec((64, 16), lambda i: (0, 0)),
        ],
        out_specs=pl.BlockSpec((_B, 16), lambda i: (i, 0)),
        out_shape=jax.ShapeDtypeStruct((NPAD, 16), jnp.float32),
    )(p, p, h1p, dinv, b1, W2)


def _final_body(q0_ref, q1_ref, h_ref, dinv_ref, b2_ref, out_ref):
    o = dinv_ref[...] * (q0_ref[0] + q1_ref[0] + h_ref[...]) + b2_ref[...]
    m = jnp.max(o, axis=1, keepdims=True)
    lse = jnp.log(jnp.sum(jnp.exp(o - m), axis=1, keepdims=True)) + m
    out_ref[...] = o - lse


def _final(q, h2p, dinv, b2):
    return pl.pallas_call(
        _final_body,
        grid=(N // _B,),
        in_specs=[
            pl.BlockSpec((1, _B, 16), lambda i: (0, i, 0)),
            pl.BlockSpec((1, _B, 16), lambda i: (1, i, 0)),
            pl.BlockSpec((_B, 16), lambda i: (i, 0)),
            pl.BlockSpec((_B, 1), lambda i: (i, 0)),
            pl.BlockSpec((1, 16), lambda i: (0, 0)),
        ],
        out_specs=pl.BlockSpec((_B, 16), lambda i: (i, 0)),
        out_shape=jax.ShapeDtypeStruct((N, 16), jnp.float32),
    )(q, q, h2p, dinv, b2)


def kernel(x, edge_index, W1, b1, W2, b2):
    pad = jnp.full((NW, EPW_PAD - EPW), N, jnp.int32)
    pad_d = jnp.full((NW, EPW_PAD - EPW), NPAD - 1, jnp.int32)
    src = jnp.concatenate(
        [edge_index[0].reshape(NW, EPW), pad], axis=1).reshape(NW, NCHUNK, K)
    dst_flat = jnp.concatenate(
        [edge_index[1].reshape(NW, EPW), pad_d], axis=1)
    dst = dst_flat.reshape(NW, NCHUNK, K)

    zeros64 = jnp.zeros((NPAD, 64), jnp.float32)
    zeros16 = jnp.zeros((NPAD, 16), jnp.float32)
    ones_k = jnp.ones((K, 16), jnp.float32)

    degp = _deg(dst, ones_k, zeros16)
    h1p, dinv = _mm1(x, W1, degp)
    p = _prop64(h1p, zeros64, src, dst)
    h2p = _mm2(p, h1p, dinv, b1.reshape(1, 64), W2)
    q = _prop16(h2p, zeros16, src, dst)
    return _final(q, h2p, dinv, b2.reshape(1, 16))
